# baseline (speedup 1.0000x reference)
import functools

import jax
import jax.numpy as jnp
from jax import lax
from jax.experimental import pallas as pl
from jax.experimental.pallas import tpu as pltpu
from jax.experimental.pallas import tpu_sc as plsc

HID = 256
HALF = 128
N_MESH = 10000
N_GRID = 10000
N_EDGE = 160000
NPAD = 10112
NSUB = 16
ROWS_PER_SUB = NPAD // NSUB
CH1 = 64
SUB_CH1 = 158
EP1 = CH1 * SUB_CH1 * NSUB
CH2 = 128
SUB_CH2 = 80
EP2 = CH2 * SUB_CH2 * NSUB
EB = 800
RB = 400


def _proj_body(mesh_f, grid_f, mesh_h, w1s, w1d, tmp_o, tgp_o, tmesh_o):
    tmp_o[0] = jnp.dot(mesh_f[...], w1s[...],
                       preferred_element_type=jnp.float32)
    tgp_o[0] = jnp.dot(grid_f[...], w1d[...],
                       preferred_element_type=jnp.float32)
    tmesh_o[0] = mesh_h[...]


def _proj_tables(mesh, grid, w1s, w1d):
    nb = N_MESH // RB
    return pl.pallas_call(
        _proj_body,
        grid=(2, nb),
        in_specs=[
            pl.BlockSpec((RB, HID), lambda c, r: (r, 0)),
            pl.BlockSpec((RB, HID), lambda c, r: (r, 0)),
            pl.BlockSpec((RB, HALF), lambda c, r: (r, c)),
            pl.BlockSpec((HID, HALF), lambda c, r: (0, c)),
            pl.BlockSpec((HID, HALF), lambda c, r: (0, c)),
        ],
        out_specs=[
            pl.BlockSpec((1, RB, HALF), lambda c, r: (c, r, 0)),
            pl.BlockSpec((1, RB, HALF), lambda c, r: (c, r, 0)),
            pl.BlockSpec((1, RB, HALF), lambda c, r: (c, r, 0)),
        ],
        out_shape=[
            jax.ShapeDtypeStruct((2, NPAD, HALF), jnp.float32),
            jax.ShapeDtypeStruct((2, NPAD, HALF), jnp.float32),
            jax.ShapeDtypeStruct((2, NPAD, HALF), jnp.float32),
        ],
    )(mesh, grid, mesh, w1s, w1d)


def _sc_gather_body(tmp_h, tgp_h, tmesh_h, srcp_h, dstp_h, dstw_h, zeros_h,
                    mp_o, gp_o, aggs_o,
                    idx_s0, idx_s1, idx_d0, idx_d1, idx_w0, idx_w1,
                    bmp0, bmp1, bgp0, bgp1, bme0, bme1, accum,
                    sem_i0, sem_i1, sem_g0, sem_g1, sem_w0, sem_w1,
                    sem_s0, sem_s1):
    c = lax.axis_index("c")
    s = lax.axis_index("s")
    cb = SUB_CH1 * s
    idx_s = [idx_s0, idx_s1]
    idx_d = [idx_d0, idx_d1]
    idx_w = [idx_w0, idx_w1]
    bmp = [bmp0, bmp1]
    bgp = [bgp0, bgp1]
    bme = [bme0, bme1]
    sem_i = [sem_i0, sem_i1]
    sem_g = [sem_g0, sem_g1]
    sem_w = [sem_w0, sem_w1]
    sem_s = [sem_s0, sem_s1]

    pltpu.sync_copy(zeros_h, accum.at[pl.ds(s * ROWS_PER_SUB, ROWS_PER_SUB)])
    plsc.subcore_barrier()

    def issue_idx(b, i):
        base = (cb + i) * CH1
        pltpu.async_copy(srcp_h.at[pl.ds(c * EP1 + base, CH1)], idx_s[b],
                         sem_i[b])
        pltpu.async_copy(dstp_h.at[pl.ds(c * EP1 + base, CH1)], idx_d[b],
                         sem_i[b])
        pltpu.async_copy(dstw_h.at[pl.ds(base, CH1)], idx_w[b].at[0], sem_i[b])

    def wait_idx(b):
        pltpu.make_async_copy(srcp_h.at[pl.ds(0, CH1)], idx_s[b],
                              sem_i[b]).wait()
        pltpu.make_async_copy(dstp_h.at[pl.ds(0, CH1)], idx_d[b],
                              sem_i[b]).wait()
        pltpu.make_async_copy(dstw_h.at[pl.ds(0, CH1)], idx_w[b].at[0],
                              sem_i[b]).wait()

    def wait_writes(b):
        pltpu.make_async_copy(bmp[b], mp_o.at[0, pl.ds(0, CH1)],
                              sem_w[b]).wait()
        pltpu.make_async_copy(bgp[b], gp_o.at[0, pl.ds(0, CH1)],
                              sem_w[b]).wait()
        pltpu.make_async_copy(bme[b], accum.at[idx_w[b].at[0]],
                              sem_s[b]).wait()

    def wait_gathers(b):
        pltpu.make_async_copy(tmp_h.at[idx_s[b]], bmp[b], sem_g[b]).wait()
        pltpu.make_async_copy(tgp_h.at[idx_d[b]], bgp[b], sem_g[b]).wait()
        pltpu.make_async_copy(tmesh_h.at[idx_s[b]], bme[b], sem_g[b]).wait()

    def drain_prev(b, i):
        wait_gathers(b)
        base = (cb + i - 1) * CH1
        pltpu.async_copy(bmp[b], mp_o.at[c, pl.ds(base, CH1)], sem_w[b])
        pltpu.async_copy(bgp[b], gp_o.at[c, pl.ds(base, CH1)], sem_w[b])
        pltpu.async_copy(bme[b], accum.at[idx_w[b].at[0]], sem_s[b], add=True)

    issue_idx(0, 0)
    issue_idx(1, 1)

    def pair(g, carry):
        for b in range(2):
            i = 2 * g + b
            wait_idx(b)

            @pl.when(g >= 1)
            def _free_slot():
                wait_writes(b)

            pltpu.async_copy(tmp_h.at[idx_s[b]], bmp[b], sem_g[b])
            pltpu.async_copy(tgp_h.at[idx_d[b]], bgp[b], sem_g[b])
            pltpu.async_copy(tmesh_h.at[idx_s[b]], bme[b], sem_g[b])

            if b == 1:
                drain_prev(0, i)
                @pl.when(g < SUB_CH1 // 2 - 1)
                def _pf():
                    issue_idx(0, i + 1)
            else:
                @pl.when(g >= 1)
                def _dp():
                    drain_prev(1, i)
                    issue_idx(1, i + 1)
        return carry

    lax.fori_loop(0, SUB_CH1 // 2, pair, 0)
    wait_gathers(1)
    basel = (cb + SUB_CH1 - 1) * CH1
    pltpu.async_copy(bmp[1], mp_o.at[c, pl.ds(basel, CH1)], sem_w[1])
    pltpu.async_copy(bgp[1], gp_o.at[c, pl.ds(basel, CH1)], sem_w[1])
    pltpu.async_copy(bme[1], accum.at[idx_w[1].at[0]], sem_s[1], add=True)
    wait_writes(0)
    wait_writes(1)
    plsc.subcore_barrier()
    r0 = s * ROWS_PER_SUB
    pltpu.sync_copy(accum.at[pl.ds(r0, ROWS_PER_SUB)],
                    aggs_o.at[c, pl.ds(r0, ROWS_PER_SUB)])


def _sc_gather(tmp, tgp, tmesh, srcp, dstp, dstw, zeros):
    mesh = plsc.VectorSubcoreMesh(core_axis_name="c", subcore_axis_name="s")
    f = functools.partial(
        pl.kernel, _sc_gather_body, mesh=mesh,
        out_type=[
            jax.ShapeDtypeStruct((2, EP1, HALF), jnp.float32),
            jax.ShapeDtypeStruct((2, EP1, HALF), jnp.float32),
            jax.ShapeDtypeStruct((2, NPAD, HALF), jnp.float32),
        ],
        scratch_types=[
            pltpu.VMEM((CH1,), jnp.int32),
            pltpu.VMEM((CH1,), jnp.int32),
            pltpu.VMEM((CH1,), jnp.int32),
            pltpu.VMEM((CH1,), jnp.int32),
            pltpu.VMEM((1, CH1), jnp.int32),
            pltpu.VMEM((1, CH1), jnp.int32),
            pltpu.VMEM((CH1, HALF), jnp.float32),
            pltpu.VMEM((CH1, HALF), jnp.float32),
            pltpu.VMEM((CH1, HALF), jnp.float32),
            pltpu.VMEM((CH1, HALF), jnp.float32),
            pltpu.VMEM((CH1, HALF), jnp.float32),
            pltpu.VMEM((CH1, HALF), jnp.float32),
            pltpu.VMEM_SHARED((NPAD, HALF), jnp.float32),
            pltpu.SemaphoreType.DMA,
            pltpu.SemaphoreType.DMA,
            pltpu.SemaphoreType.DMA,
            pltpu.SemaphoreType.DMA,
            pltpu.SemaphoreType.DMA,
            pltpu.SemaphoreType.DMA,
            pltpu.SemaphoreType.DMA,
            pltpu.SemaphoreType.DMA,
        ],
    )()
    return f(tmp, tgp, tmesh, srcp, dstp, dstw, zeros)


def _edge_body(e_r, mp_r, gp_r, w1e_r, w2_r, b1_r, b2_r, g_r, bt_r, out_r):
    e_blk = e_r[...]
    f32 = jnp.float32
    gath = jnp.concatenate([mp_r[0] + gp_r[0], mp_r[1] + gp_r[1]], axis=1)
    pre = jnp.dot(e_blk.astype(jnp.bfloat16), w1e_r[...].astype(jnp.bfloat16),
                  preferred_element_type=f32)
    pre = pre + gath + b1_r[...]
    h = pre * jax.nn.sigmoid(pre)
    h2 = jnp.dot(h.astype(jnp.bfloat16), w2_r[...].astype(jnp.bfloat16),
                 preferred_element_type=f32) + b2_r[...]
    mu = jnp.mean(h2, axis=-1, keepdims=True)
    var = jnp.mean((h2 - mu) ** 2, axis=-1, keepdims=True)
    ef = (h2 - mu) / jnp.sqrt(var + 1e-5) * g_r[...] + bt_r[...] + e_blk
    out_r[0] = ef[:, :HALF]
    out_r[1] = ef[:, HALF:]


def _edge_mlp(e, mp_g, gp_g, w1e, w2, b1, b2, g, bt):
    nb = N_EDGE // EB
    return pl.pallas_call(
        _edge_body,
        grid=(nb,),
        in_specs=[
            pl.BlockSpec((EB, HID), lambda k: (k, 0)),
            pl.BlockSpec((2, EB, HALF), lambda k: (0, k, 0)),
            pl.BlockSpec((2, EB, HALF), lambda k: (0, k, 0)),
            pl.BlockSpec((HID, HID), lambda k: (0, 0)),
            pl.BlockSpec((HID, HID), lambda k: (0, 0)),
            pl.BlockSpec((1, HID), lambda k: (0, 0)),
            pl.BlockSpec((1, HID), lambda k: (0, 0)),
            pl.BlockSpec((1, HID), lambda k: (0, 0)),
            pl.BlockSpec((1, HID), lambda k: (0, 0)),
        ],
        out_specs=pl.BlockSpec((2, EB, HALF), lambda k: (0, k, 0)),
        out_shape=jax.ShapeDtypeStruct((2, EP2, HALF), jnp.float32),
    )(e, mp_g, gp_g, w1e, w2, b1, b2, g, bt)


def _sc_scatter_body(ef_h, dstw_h, zeros_h, agge_o,
                     idx_w0, idx_w1, buf0, buf1, accum,
                     sem_i0, sem_i1, sem_l0, sem_l1, sem_s0, sem_s1):
    c = lax.axis_index("c")
    s = lax.axis_index("s")
    cb = SUB_CH2 * s
    idx_w = [idx_w0, idx_w1]
    buf = [buf0, buf1]
    sem_i = [sem_i0, sem_i1]
    sem_l = [sem_l0, sem_l1]
    sem_s = [sem_s0, sem_s1]

    pltpu.sync_copy(zeros_h, accum.at[pl.ds(s * ROWS_PER_SUB, ROWS_PER_SUB)])
    plsc.subcore_barrier()

    def issue(b, i):
        base = (cb + i) * CH2
        pltpu.async_copy(dstw_h.at[pl.ds(base, CH2)], idx_w[b].at[0], sem_i[b])
        pltpu.async_copy(ef_h.at[c, pl.ds(base, CH2)], buf[b], sem_l[b])

    def wait_loads(b):
        pltpu.make_async_copy(dstw_h.at[pl.ds(0, CH2)], idx_w[b].at[0],
                              sem_i[b]).wait()
        pltpu.make_async_copy(ef_h.at[0, pl.ds(0, CH2)], buf[b],
                              sem_l[b]).wait()

    def wait_scatter(b):
        pltpu.make_async_copy(buf[b], accum.at[idx_w[b].at[0]],
                              sem_s[b]).wait()

    issue(0, 0)
    issue(1, 1)

    def pair(g, carry):
        for b in range(2):
            i = 2 * g + b
            wait_loads(b)
            pltpu.async_copy(buf[b], accum.at[idx_w[b].at[0]], sem_s[b],
                             add=True)
            if b == 1:
                wait_scatter(0)
                @pl.when(g < SUB_CH2 // 2 - 1)
                def _pf():
                    issue(0, i + 1)
            else:
                @pl.when(g >= 1)
                def _dp():
                    wait_scatter(1)
                    issue(1, i + 1)
        return carry

    lax.fori_loop(0, SUB_CH2 // 2, pair, 0)
    wait_scatter(1)
    plsc.subcore_barrier()
    r0 = s * ROWS_PER_SUB
    pltpu.sync_copy(accum.at[pl.ds(r0, ROWS_PER_SUB)],
                    agge_o.at[c, pl.ds(r0, ROWS_PER_SUB)])


def _sc_scatter(ef, dstw, zeros):
    mesh = plsc.VectorSubcoreMesh(core_axis_name="c", subcore_axis_name="s")
    f = functools.partial(
        pl.kernel, _sc_scatter_body, mesh=mesh,
        out_type=jax.ShapeDtypeStruct((2, NPAD, HALF), jnp.float32),
        scratch_types=[
            pltpu.VMEM((1, CH2), jnp.int32),
            pltpu.VMEM((1, CH2), jnp.int32),
            pltpu.VMEM((CH2, HALF), jnp.float32),
            pltpu.VMEM((CH2, HALF), jnp.float32),
            pltpu.VMEM_SHARED((NPAD, HALF), jnp.float32),
            pltpu.SemaphoreType.DMA,
            pltpu.SemaphoreType.DMA,
            pltpu.SemaphoreType.DMA,
            pltpu.SemaphoreType.DMA,
            pltpu.SemaphoreType.DMA,
            pltpu.SemaphoreType.DMA,
        ],
    )()
    return f(ef, dstw, zeros)


def _node_body(as_r, ae_r, grid_r, w1s_r, w1e_r, w2_r, b1_r, b2_r, g_r, bt_r,
               out_r):
    a_s = jnp.concatenate([as_r[0], as_r[1]], axis=1)
    a_e = jnp.concatenate([ae_r[0], ae_r[1]], axis=1)
    n1 = (jnp.dot(a_s, w1s_r[...], preferred_element_type=jnp.float32)
          + jnp.dot(a_e, w1e_r[...], preferred_element_type=jnp.float32)
          + b1_r[...])
    h = n1 * jax.nn.sigmoid(n1)
    n2 = jnp.dot(h, w2_r[...], preferred_element_type=jnp.float32) + b2_r[...]
    mu = jnp.mean(n2, axis=-1, keepdims=True)
    var = jnp.mean((n2 - mu) ** 2, axis=-1, keepdims=True)
    out_r[...] = (n2 - mu) / jnp.sqrt(var + 1e-5) * g_r[...] + bt_r[...] \
        + grid_r[...]


def _node_mlp(agg_s, agg_e, grid, w1s, w1e, w2, b1, b2, g, bt):
    nb = N_GRID // RB
    return pl.pallas_call(
        _node_body,
        grid=(nb,),
        in_specs=[
            pl.BlockSpec((2, RB, HALF), lambda r: (0, r, 0)),
            pl.BlockSpec((2, RB, HALF), lambda r: (0, r, 0)),
            pl.BlockSpec((RB, HID), lambda r: (r, 0)),
            pl.BlockSpec((HID, HID), lambda r: (0, 0)),
            pl.BlockSpec((HID, HID), lambda r: (0, 0)),
            pl.BlockSpec((HID, HID), lambda r: (0, 0)),
            pl.BlockSpec((1, HID), lambda r: (0, 0)),
            pl.BlockSpec((1, HID), lambda r: (0, 0)),
            pl.BlockSpec((1, HID), lambda r: (0, 0)),
            pl.BlockSpec((1, HID), lambda r: (0, 0)),
        ],
        out_specs=pl.BlockSpec((RB, HID), lambda r: (r, 0)),
        out_shape=jax.ShapeDtypeStruct((N_GRID, HID), jnp.float32),
    )(agg_s, agg_e, grid, w1s, w1e, w2, b1, b2, g, bt)


def kernel(mesh2grid_edge_features, grid_node_features, mesh_node_features,
           mesh2graph_edge_indices_src, mesh2graph_edge_indices_dst,
           eW1, eb1, eW2, eb2, eg, ebt, nW1, nb1, nW2, nb2, ng, nbt):
    e = mesh2grid_edge_features
    grid = grid_node_features
    mesh = mesh_node_features
    src = mesh2graph_edge_indices_src.astype(jnp.int32)
    dst = mesh2graph_edge_indices_dst.astype(jnp.int32)

    w1e, w1s, w1d = eW1[:HID], eW1[HID:2 * HID], eW1[2 * HID:]
    nw1s, nw1e = nW1[:HID], nW1[HID:]
    r2 = lambda v: v.reshape(1, HID)

    pad1 = EP1 - N_EDGE
    srcp = jnp.concatenate([src, jnp.zeros((pad1,), jnp.int32),
                            src + NPAD, jnp.zeros((pad1,), jnp.int32)])
    dstp = jnp.concatenate([dst, jnp.full((pad1,), N_GRID, jnp.int32),
                            dst + NPAD, jnp.full((pad1,), NPAD + N_GRID,
                                                 jnp.int32)])
    dstw = jnp.concatenate([dst, jnp.full((EP2 - N_EDGE,), N_GRID,
                                          jnp.int32)])
    zeros = jnp.zeros((ROWS_PER_SUB, HALF), jnp.float32)

    tmp, tgp, tmesh = _proj_tables(mesh, grid, w1s, w1d)
    tmp = tmp.reshape(2 * NPAD, HALF)
    tgp = tgp.reshape(2 * NPAD, HALF)
    tmesh = tmesh.reshape(2 * NPAD, HALF)

    mp_g, gp_g, agg_s = _sc_gather(tmp, tgp, tmesh, srcp, dstp, dstw, zeros)
    ef = _edge_mlp(e, mp_g, gp_g, w1e, eW2, r2(eb1), r2(eb2), r2(eg), r2(ebt))
    agg_e = _sc_scatter(ef, dstw, zeros)
    return _node_mlp(agg_s, agg_e, grid, nw1s, nw1e, nW2,
                     r2(nb1), r2(nb2), r2(ng), r2(nbt))


# speedup vs baseline: 1.0915x; 1.0915x over previous
import functools

import jax
import jax.numpy as jnp
from jax import lax
from jax.experimental import pallas as pl
from jax.experimental.pallas import tpu as pltpu
from jax.experimental.pallas import tpu_sc as plsc

HID = 256
HALF = 128
QUART = 64
N_MESH = 10000
N_GRID = 10000
N_EDGE = 160000
NPAD = 10112
NSUB = 16
ROWS_PER_SUB = NPAD // NSUB
CH1M = 48
CH1T = 96
SUB_CH1 = 106
EP1 = CH1T * SUB_CH1 * NSUB
CH2 = 128
SUB_CH2 = 80
EP2 = CH2 * SUB_CH2 * NSUB
EB = 800
RB = 400


def _pack_bf16(x):
    lo = x[:, :HALF].astype(jnp.bfloat16)
    hi = x[:, HALF:].astype(jnp.bfloat16)
    lo32 = jax.lax.bitcast_convert_type(lo, jnp.int16).astype(jnp.int32)
    hi32 = jax.lax.bitcast_convert_type(hi, jnp.int16).astype(jnp.int32)
    return (lo32 & 0xFFFF) | (hi32 << 16)


def _unpack_bf16(x):
    lo = jax.lax.bitcast_convert_type(x << 16, jnp.float32)
    hi = jax.lax.bitcast_convert_type((x >> 16) << 16, jnp.float32)
    return lo, hi


def _proj_body(mesh_f, grid_f, w1s, w1d, tmp_o, tgp_o):
    bf = jnp.bfloat16
    tmp_o[...] = _pack_bf16(jnp.dot(mesh_f[...].astype(bf),
                                    w1s[...].astype(bf),
                                    preferred_element_type=jnp.float32))
    tgp_o[...] = _pack_bf16(jnp.dot(grid_f[...].astype(bf),
                                    w1d[...].astype(bf),
                                    preferred_element_type=jnp.float32))


def _proj_tables(mesh, grid, w1s, w1d):
    nb = N_MESH // RB
    return pl.pallas_call(
        _proj_body,
        grid=(nb,),
        in_specs=[
            pl.BlockSpec((RB, HID), lambda r: (r, 0)),
            pl.BlockSpec((RB, HID), lambda r: (r, 0)),
            pl.BlockSpec((HID, HID), lambda r: (0, 0)),
            pl.BlockSpec((HID, HID), lambda r: (0, 0)),
        ],
        out_specs=[
            pl.BlockSpec((RB, HALF), lambda r: (r, 0)),
            pl.BlockSpec((RB, HALF), lambda r: (r, 0)),
        ],
        out_shape=[
            jax.ShapeDtypeStruct((NPAD, HALF), jnp.int32),
            jax.ShapeDtypeStruct((NPAD, HALF), jnp.int32),
        ],
    )(mesh, grid, w1s, w1d)


def _mesh_halves_body(mesh_h, tmesh_o):
    tmesh_o[0] = mesh_h[...]


def _mesh_halves(mesh):
    nb = N_MESH // RB
    return pl.pallas_call(
        _mesh_halves_body,
        grid=(2, nb),
        in_specs=[pl.BlockSpec((RB, HALF), lambda c, r: (r, c))],
        out_specs=pl.BlockSpec((1, RB, HALF), lambda c, r: (c, r, 0)),
        out_shape=jax.ShapeDtypeStruct((2, NPAD, HALF), jnp.float32),
    )(mesh)


def _sc_gather_body(tmp_h, tgp_h, tmesh_h, srcm_h, dstm_h, srcp_h, dstw_h,
                    zeros_h,
                    mp_o, gp_o, aggs_o,
                    idx_s0, idx_s1, idx_d0, idx_d1, idx_t0, idx_t1,
                    idx_w0, idx_w1,
                    bmp0, bmp1, bgp0, bgp1, bme0, bme1, accum,
                    sem_i0, sem_i1, sem_g0, sem_g1, sem_w0, sem_w1,
                    sem_s0, sem_s1):
    c = lax.axis_index("c")
    s = lax.axis_index("s")
    w = c * NSUB + s
    mb = SUB_CH1 * w
    tb = SUB_CH1 * s
    idx_s = [idx_s0, idx_s1]
    idx_d = [idx_d0, idx_d1]
    idx_t = [idx_t0, idx_t1]
    idx_w = [idx_w0, idx_w1]
    bmp = [bmp0, bmp1]
    bgp = [bgp0, bgp1]
    bme = [bme0, bme1]
    sem_i = [sem_i0, sem_i1]
    sem_g = [sem_g0, sem_g1]
    sem_w = [sem_w0, sem_w1]
    sem_s = [sem_s0, sem_s1]

    pltpu.sync_copy(zeros_h, accum.at[pl.ds(s * ROWS_PER_SUB, ROWS_PER_SUB)])
    plsc.subcore_barrier()

    def issue_idx(b, i):
        basem = (mb + i) * CH1M
        baset = (tb + i) * CH1T
        pltpu.async_copy(srcm_h.at[pl.ds(basem, CH1M)], idx_s[b], sem_i[b])
        pltpu.async_copy(dstm_h.at[pl.ds(basem, CH1M)], idx_d[b], sem_i[b])
        pltpu.async_copy(srcp_h.at[pl.ds(c * EP1 + baset, CH1T)], idx_t[b],
                         sem_i[b])
        pltpu.async_copy(dstw_h.at[pl.ds(baset, CH1T)], idx_w[b].at[0],
                         sem_i[b])

    def wait_idx(b):
        pltpu.make_async_copy(srcm_h.at[pl.ds(0, CH1M)], idx_s[b],
                              sem_i[b]).wait()
        pltpu.make_async_copy(dstm_h.at[pl.ds(0, CH1M)], idx_d[b],
                              sem_i[b]).wait()
        pltpu.make_async_copy(srcp_h.at[pl.ds(0, CH1T)], idx_t[b],
                              sem_i[b]).wait()
        pltpu.make_async_copy(dstw_h.at[pl.ds(0, CH1T)], idx_w[b].at[0],
                              sem_i[b]).wait()

    def wait_writes(b):
        pltpu.make_async_copy(bmp[b], mp_o.at[pl.ds(0, CH1M)],
                              sem_w[b]).wait()
        pltpu.make_async_copy(bgp[b], gp_o.at[pl.ds(0, CH1M)],
                              sem_w[b]).wait()
        pltpu.make_async_copy(bme[b], accum.at[idx_w[b].at[0]],
                              sem_s[b]).wait()

    def wait_gathers(b):
        pltpu.make_async_copy(tmp_h.at[idx_s[b]], bmp[b], sem_g[b]).wait()
        pltpu.make_async_copy(tgp_h.at[idx_d[b]], bgp[b], sem_g[b]).wait()
        pltpu.make_async_copy(tmesh_h.at[idx_t[b]], bme[b], sem_g[b]).wait()

    def drain_prev(b, i):
        wait_gathers(b)
        basem = (mb + i - 1) * CH1M
        pltpu.async_copy(bmp[b], mp_o.at[pl.ds(basem, CH1M)], sem_w[b])
        pltpu.async_copy(bgp[b], gp_o.at[pl.ds(basem, CH1M)], sem_w[b])
        pltpu.async_copy(bme[b], accum.at[idx_w[b].at[0]], sem_s[b], add=True)

    issue_idx(0, 0)
    issue_idx(1, 1)

    def pair(g, carry):
        for b in range(2):
            i = 2 * g + b
            wait_idx(b)

            @pl.when(g >= 1)
            def _free_slot():
                wait_writes(b)

            pltpu.async_copy(tmp_h.at[idx_s[b]], bmp[b], sem_g[b])
            pltpu.async_copy(tgp_h.at[idx_d[b]], bgp[b], sem_g[b])
            pltpu.async_copy(tmesh_h.at[idx_t[b]], bme[b], sem_g[b])

            if b == 1:
                drain_prev(0, i)
                @pl.when(g < SUB_CH1 // 2 - 1)
                def _pf():
                    issue_idx(0, i + 1)
            else:
                @pl.when(g >= 1)
                def _dp():
                    drain_prev(1, i)
                    issue_idx(1, i + 1)
        return carry

    lax.fori_loop(0, SUB_CH1 // 2, pair, 0)
    wait_gathers(1)
    basel = (mb + SUB_CH1 - 1) * CH1M
    pltpu.async_copy(bmp[1], mp_o.at[pl.ds(basel, CH1M)], sem_w[1])
    pltpu.async_copy(bgp[1], gp_o.at[pl.ds(basel, CH1M)], sem_w[1])
    pltpu.async_copy(bme[1], accum.at[idx_w[1].at[0]], sem_s[1], add=True)
    wait_writes(0)
    wait_writes(1)
    plsc.subcore_barrier()
    r0 = s * ROWS_PER_SUB
    pltpu.sync_copy(accum.at[pl.ds(r0, ROWS_PER_SUB)],
                    aggs_o.at[c, pl.ds(r0, ROWS_PER_SUB)])


def _sc_gather(tmp, tgp, tmesh, srcm, dstm, srcp, dstw, zeros):
    mesh = plsc.VectorSubcoreMesh(core_axis_name="c", subcore_axis_name="s")
    f = functools.partial(
        pl.kernel, _sc_gather_body, mesh=mesh,
        out_type=[
            jax.ShapeDtypeStruct((EP1, HALF), jnp.int32),
            jax.ShapeDtypeStruct((EP1, HALF), jnp.int32),
            jax.ShapeDtypeStruct((2, NPAD, HALF), jnp.float32),
        ],
        scratch_types=[
            pltpu.VMEM((CH1M,), jnp.int32),
            pltpu.VMEM((CH1M,), jnp.int32),
            pltpu.VMEM((CH1M,), jnp.int32),
            pltpu.VMEM((CH1M,), jnp.int32),
            pltpu.VMEM((CH1T,), jnp.int32),
            pltpu.VMEM((CH1T,), jnp.int32),
            pltpu.VMEM((1, CH1T), jnp.int32),
            pltpu.VMEM((1, CH1T), jnp.int32),
            pltpu.VMEM((CH1M, HALF), jnp.int32),
            pltpu.VMEM((CH1M, HALF), jnp.int32),
            pltpu.VMEM((CH1M, HALF), jnp.int32),
            pltpu.VMEM((CH1M, HALF), jnp.int32),
            pltpu.VMEM((CH1T, HALF), jnp.float32),
            pltpu.VMEM((CH1T, HALF), jnp.float32),
            pltpu.VMEM_SHARED((NPAD, HALF), jnp.float32),
            pltpu.SemaphoreType.DMA,
            pltpu.SemaphoreType.DMA,
            pltpu.SemaphoreType.DMA,
            pltpu.SemaphoreType.DMA,
            pltpu.SemaphoreType.DMA,
            pltpu.SemaphoreType.DMA,
            pltpu.SemaphoreType.DMA,
            pltpu.SemaphoreType.DMA,
        ],
    )()
    return f(tmp, tgp, tmesh, srcm, dstm, srcp, dstw, zeros)


def _edge_body(e_r, mp_r, gp_r, w1e_r, w2_r, b1_r, b2_r, g_r, bt_r, out_r):
    e_blk = e_r[...]
    f32 = jnp.float32
    mpl, mph = _unpack_bf16(mp_r[...])
    gpl, gph = _unpack_bf16(gp_r[...])
    gath = jnp.concatenate([mpl + gpl, mph + gph], axis=1)
    pre = jnp.dot(e_blk.astype(jnp.bfloat16), w1e_r[...].astype(jnp.bfloat16),
                  preferred_element_type=f32)
    pre = pre + gath + b1_r[...]
    h = pre * jax.nn.sigmoid(pre)
    h2 = jnp.dot(h.astype(jnp.bfloat16), w2_r[...].astype(jnp.bfloat16),
                 preferred_element_type=f32) + b2_r[...]
    mu = jnp.mean(h2, axis=-1, keepdims=True)
    var = jnp.mean((h2 - mu) ** 2, axis=-1, keepdims=True)
    ef = (h2 - mu) / jnp.sqrt(var + 1e-5) * g_r[...] + bt_r[...] + e_blk
    out_r[0] = ef[:, :HALF]
    out_r[1] = ef[:, HALF:]


def _edge_mlp(e, mp_g, gp_g, w1e, w2, b1, b2, g, bt):
    nb = N_EDGE // EB
    return pl.pallas_call(
        _edge_body,
        grid=(nb,),
        in_specs=[
            pl.BlockSpec((EB, HID), lambda k: (k, 0)),
            pl.BlockSpec((EB, HALF), lambda k: (k, 0)),
            pl.BlockSpec((EB, HALF), lambda k: (k, 0)),
            pl.BlockSpec((HID, HID), lambda k: (0, 0)),
            pl.BlockSpec((HID, HID), lambda k: (0, 0)),
            pl.BlockSpec((1, HID), lambda k: (0, 0)),
            pl.BlockSpec((1, HID), lambda k: (0, 0)),
            pl.BlockSpec((1, HID), lambda k: (0, 0)),
            pl.BlockSpec((1, HID), lambda k: (0, 0)),
        ],
        out_specs=pl.BlockSpec((2, EB, HALF), lambda k: (0, k, 0)),
        out_shape=jax.ShapeDtypeStruct((2, EP2, HALF), jnp.float32),
    )(e, mp_g, gp_g, w1e, w2, b1, b2, g, bt)


def _sc_scatter_body(ef_h, dstw_h, zeros_h, agge_o,
                     idx_w0, idx_w1, buf0, buf1, accum,
                     sem_i0, sem_i1, sem_l0, sem_l1, sem_s0, sem_s1):
    c = lax.axis_index("c")
    s = lax.axis_index("s")
    cb = SUB_CH2 * s
    idx_w = [idx_w0, idx_w1]
    buf = [buf0, buf1]
    sem_i = [sem_i0, sem_i1]
    sem_l = [sem_l0, sem_l1]
    sem_s = [sem_s0, sem_s1]

    pltpu.sync_copy(zeros_h, accum.at[pl.ds(s * ROWS_PER_SUB, ROWS_PER_SUB)])
    plsc.subcore_barrier()

    def issue(b, i):
        base = (cb + i) * CH2
        pltpu.async_copy(dstw_h.at[pl.ds(base, CH2)], idx_w[b].at[0], sem_i[b])
        pltpu.async_copy(ef_h.at[c, pl.ds(base, CH2)], buf[b], sem_l[b])

    def wait_loads(b):
        pltpu.make_async_copy(dstw_h.at[pl.ds(0, CH2)], idx_w[b].at[0],
                              sem_i[b]).wait()
        pltpu.make_async_copy(ef_h.at[0, pl.ds(0, CH2)], buf[b],
                              sem_l[b]).wait()

    def wait_scatter(b):
        pltpu.make_async_copy(buf[b], accum.at[idx_w[b].at[0]],
                              sem_s[b]).wait()

    issue(0, 0)
    issue(1, 1)

    def pair(g, carry):
        for b in range(2):
            i = 2 * g + b
            wait_loads(b)
            pltpu.async_copy(buf[b], accum.at[idx_w[b].at[0]], sem_s[b],
                             add=True)
            if b == 1:
                wait_scatter(0)
                @pl.when(g < SUB_CH2 // 2 - 1)
                def _pf():
                    issue(0, i + 1)
            else:
                @pl.when(g >= 1)
                def _dp():
                    wait_scatter(1)
                    issue(1, i + 1)
        return carry

    lax.fori_loop(0, SUB_CH2 // 2, pair, 0)
    wait_scatter(1)
    plsc.subcore_barrier()
    r0 = s * ROWS_PER_SUB
    pltpu.sync_copy(accum.at[pl.ds(r0, ROWS_PER_SUB)],
                    agge_o.at[c, pl.ds(r0, ROWS_PER_SUB)])


def _sc_scatter(ef, dstw, zeros):
    mesh = plsc.VectorSubcoreMesh(core_axis_name="c", subcore_axis_name="s")
    f = functools.partial(
        pl.kernel, _sc_scatter_body, mesh=mesh,
        out_type=jax.ShapeDtypeStruct((2, NPAD, HALF), jnp.float32),
        scratch_types=[
            pltpu.VMEM((1, CH2), jnp.int32),
            pltpu.VMEM((1, CH2), jnp.int32),
            pltpu.VMEM((CH2, HALF), jnp.float32),
            pltpu.VMEM((CH2, HALF), jnp.float32),
            pltpu.VMEM_SHARED((NPAD, HALF), jnp.float32),
            pltpu.SemaphoreType.DMA,
            pltpu.SemaphoreType.DMA,
            pltpu.SemaphoreType.DMA,
            pltpu.SemaphoreType.DMA,
            pltpu.SemaphoreType.DMA,
            pltpu.SemaphoreType.DMA,
        ],
    )()
    return f(ef, dstw, zeros)


def _node_body(as_r, ae_r, grid_r, w1s_r, w1e_r, w2_r, b1_r, b2_r, g_r, bt_r,
               out_r):
    a_s = jnp.concatenate([as_r[0], as_r[1]], axis=1)
    a_e = jnp.concatenate([ae_r[0], ae_r[1]], axis=1)
    n1 = (jnp.dot(a_s, w1s_r[...], preferred_element_type=jnp.float32)
          + jnp.dot(a_e, w1e_r[...], preferred_element_type=jnp.float32)
          + b1_r[...])
    h = n1 * jax.nn.sigmoid(n1)
    n2 = jnp.dot(h, w2_r[...], preferred_element_type=jnp.float32) + b2_r[...]
    mu = jnp.mean(n2, axis=-1, keepdims=True)
    var = jnp.mean((n2 - mu) ** 2, axis=-1, keepdims=True)
    out_r[...] = (n2 - mu) / jnp.sqrt(var + 1e-5) * g_r[...] + bt_r[...] \
        + grid_r[...]


def _node_mlp(agg_s, agg_e, grid, w1s, w1e, w2, b1, b2, g, bt):
    nb = N_GRID // RB
    return pl.pallas_call(
        _node_body,
        grid=(nb,),
        in_specs=[
            pl.BlockSpec((2, RB, HALF), lambda r: (0, r, 0)),
            pl.BlockSpec((2, RB, HALF), lambda r: (0, r, 0)),
            pl.BlockSpec((RB, HID), lambda r: (r, 0)),
            pl.BlockSpec((HID, HID), lambda r: (0, 0)),
            pl.BlockSpec((HID, HID), lambda r: (0, 0)),
            pl.BlockSpec((HID, HID), lambda r: (0, 0)),
            pl.BlockSpec((1, HID), lambda r: (0, 0)),
            pl.BlockSpec((1, HID), lambda r: (0, 0)),
            pl.BlockSpec((1, HID), lambda r: (0, 0)),
            pl.BlockSpec((1, HID), lambda r: (0, 0)),
        ],
        out_specs=pl.BlockSpec((RB, HID), lambda r: (r, 0)),
        out_shape=jax.ShapeDtypeStruct((N_GRID, HID), jnp.float32),
    )(agg_s, agg_e, grid, w1s, w1e, w2, b1, b2, g, bt)


def kernel(mesh2grid_edge_features, grid_node_features, mesh_node_features,
           mesh2graph_edge_indices_src, mesh2graph_edge_indices_dst,
           eW1, eb1, eW2, eb2, eg, ebt, nW1, nb1, nW2, nb2, ng, nbt):
    e = mesh2grid_edge_features
    grid = grid_node_features
    mesh = mesh_node_features
    src = mesh2graph_edge_indices_src.astype(jnp.int32)
    dst = mesh2graph_edge_indices_dst.astype(jnp.int32)

    w1e, w1s, w1d = eW1[:HID], eW1[HID:2 * HID], eW1[2 * HID:]
    nw1s, nw1e = nW1[:HID], nW1[HID:]
    r2 = lambda v: v.reshape(1, HID)

    pad1 = EP1 - N_EDGE
    srcm = jnp.concatenate([src, jnp.zeros((pad1,), jnp.int32)])
    dstm = jnp.concatenate([dst, jnp.zeros((pad1,), jnp.int32)])
    srcp = jnp.concatenate([src, jnp.zeros((pad1,), jnp.int32),
                            src + NPAD, jnp.full((pad1,), NPAD, jnp.int32)])
    dstw = jnp.concatenate([dst, jnp.full((EP2 - N_EDGE,), N_GRID,
                                          jnp.int32)])
    zeros = jnp.zeros((ROWS_PER_SUB, HALF), jnp.float32)

    tmp, tgp = _proj_tables(mesh, grid, w1s, w1d)
    tmesh = _mesh_halves(mesh).reshape(2 * NPAD, HALF)

    mp_g, gp_g, agg_s = _sc_gather(tmp, tgp, tmesh, srcm, dstm, srcp, dstw,
                                   zeros)
    ef = _edge_mlp(e, mp_g, gp_g, w1e, eW2, r2(eb1), r2(eb2), r2(eg), r2(ebt))
    agg_e = _sc_scatter(ef, dstw, zeros)
    return _node_mlp(agg_s, agg_e, grid, nw1s, nw1e, nW2,
                     r2(nb1), r2(nb2), r2(ng), r2(nbt))
